# baseline (speedup 1.0000x reference)
import functools
import jax
import jax.numpy as jnp
from jax.experimental import pallas as pl

_BLOCK = 512


def _stage_body(first, final, *refs):
    if first:
        (r_ref, cb_ref, h_ref, r2_ref, cb2_ref,
         idx_ref, rn_ref, qn_ref, ls_ref) = refs
        e_ref = qs_ref = None
    elif final:
        (e_ref, r_ref, qs_ref, cb_ref, h_ref, r2_ref,
         cb2_ref, idx_ref, qn_ref, ls_ref) = refs
        rn_ref = None
    else:
        (r_ref, qs_ref, cb_ref, h_ref, r2_ref, cb2_ref,
         idx_ref, rn_ref, qn_ref, ls_ref) = refs
    i = pl.program_id(0)
    num_codes = cb_ref.shape[0]
    r = r_ref[...]
    cb = cb_ref[...]
    s = jax.lax.dot_general(r, cb, (((1,), (1,)), ((), ())),
                            preferred_element_type=jnp.float32)
    d = r2_ref[...] - 2.0 * s + cb2_ref[...]
    md = jnp.min(d, axis=1, keepdims=True)
    iota = jax.lax.broadcasted_iota(jnp.int32, d.shape, 1)
    idx = jnp.min(jnp.where(d == md, iota, num_codes), axis=1, keepdims=True)
    idx_ref[...] = idx
    oh = (iota == idx).astype(jnp.float32)
    dn = (((1,), (0,)), ((), ()))
    oh3 = jnp.concatenate([oh, oh, oh], axis=1)
    q = jax.lax.dot_general(oh3, h_ref[...], dn,
                            preferred_element_type=jnp.float32)
    diff = q - r
    qs = q if first else qs_ref[...] + q
    if final:
        e = e_ref[...]
        qn_ref[...] = e + (qs - e)
    else:
        qn_ref[...] = qs
    if not final:
        rn_ref[...] = r - q

    @pl.when(i == 0)
    def _():
        ls_ref[...] = jnp.zeros_like(ls_ref)

    ls_ref[...] += jnp.sum(diff * diff).reshape(1, 1)


def _stage_call(first, final, n, dim, num_codes, b):
    body = functools.partial(_stage_body, first, final)
    row = lambda i: (i, 0)
    rep = lambda i: (0, 0)
    row_spec = pl.BlockSpec((b, dim), row)
    cb_spec = pl.BlockSpec((num_codes, dim), rep)
    in_specs = []
    if final:
        in_specs.append(row_spec)
    in_specs.append(row_spec)
    if not first:
        in_specs.append(row_spec)
    in_specs += [cb_spec,
                 pl.BlockSpec((3 * num_codes, dim), rep),
                 pl.BlockSpec((b, 1), row),
                 pl.BlockSpec((1, num_codes), rep)]
    out_specs = [pl.BlockSpec((b, 1), row)]
    out_shape = [jax.ShapeDtypeStruct((n, 1), jnp.int32)]
    if not final:
        out_specs.append(row_spec)
        out_shape.append(jax.ShapeDtypeStruct((n, dim), jnp.float32))
    out_specs.append(row_spec)
    out_shape.append(jax.ShapeDtypeStruct((n, dim), jnp.float32))
    out_specs.append(pl.BlockSpec((1, 1), rep))
    out_shape.append(jax.ShapeDtypeStruct((1, 1), jnp.float32))
    return pl.pallas_call(body, grid=(n // b,), in_specs=in_specs,
                          out_specs=out_specs, out_shape=out_shape)


def kernel(embeds, codebooks):
    n, dim = embeds.shape
    depth, num_codes, _ = codebooks.shape
    b = _BLOCK

    h1 = jax.lax.reduce_precision(codebooks, 8, 7)
    c1 = codebooks - h1
    h2 = jax.lax.reduce_precision(c1, 8, 7)
    h3 = c1 - h2
    hcat = jnp.concatenate([h1, h2, h3], axis=1)

    r = embeds
    qs = None
    idx_cols = []
    losses = []
    for g in range(depth):
        cb = codebooks[g]
        r2 = jnp.sum(r * r, axis=1, keepdims=True)
        cb2 = jnp.sum(cb * cb, axis=1)[None, :]
        first = g == 0
        final = g == depth - 1
        call = _stage_call(first, final, n, dim, num_codes, b)
        args = []
        if final:
            args.append(embeds)
        args.append(r)
        if not first:
            args.append(qs)
        args += [cb, hcat[g], r2, cb2]
        outs = call(*args)
        if final:
            idx, qs, ls = outs
        else:
            idx, r, qs, ls = outs
        idx_cols.append(idx)
        m = ls[0, 0] / (n * dim)
        losses.append(m + 0.25 * m)
    indices = jnp.concatenate(idx_cols, axis=1)
    loss = jnp.mean(jnp.stack(losses))
    return qs, indices, loss


# speedup vs baseline: 1.0283x; 1.0283x over previous
import functools
import jax
import jax.numpy as jnp
from jax.experimental import pallas as pl

_BLOCK = 512


def _stage_body(first, final, *refs):
    if first:
        (r_ref, cb_ref, h_ref, r2_ref, cb2_ref,
         idx_ref, rn_ref, qn_ref, ls_ref) = refs
        e_ref = qs_ref = None
    elif final:
        (e_ref, r_ref, qs_ref, cb_ref, h_ref, r2_ref,
         cb2_ref, idx_ref, qn_ref, ls_ref) = refs
        rn_ref = None
    else:
        (r_ref, qs_ref, cb_ref, h_ref, r2_ref, cb2_ref,
         idx_ref, rn_ref, qn_ref, ls_ref) = refs
    i = pl.program_id(0)
    num_codes = cb_ref.shape[0]
    r = r_ref[...]
    cb = cb_ref[...]
    s = jax.lax.dot_general(r, cb, (((1,), (1,)), ((), ())),
                            preferred_element_type=jnp.float32)
    d = r2_ref[...] - 2.0 * s + cb2_ref[...]
    md = jnp.min(d, axis=1, keepdims=True)
    iota = jax.lax.broadcasted_iota(jnp.int32, d.shape, 1)
    idx = jnp.min(jnp.where(d == md, iota, num_codes), axis=1, keepdims=True)
    idx_ref[...] = idx
    oh = (iota == idx).astype(jnp.bfloat16)
    dn = (((1,), (0,)), ((), ()))
    oh3 = jnp.concatenate([oh, oh, oh], axis=1)
    q = jax.lax.dot_general(oh3, h_ref[...], dn,
                            preferred_element_type=jnp.float32)
    diff = q - r
    qs = q if first else qs_ref[...] + q
    if final:
        e = e_ref[...]
        qn_ref[...] = e + (qs - e)
    else:
        qn_ref[...] = qs
    if not final:
        rn_ref[...] = r - q

    @pl.when(i == 0)
    def _():
        ls_ref[...] = jnp.zeros_like(ls_ref)

    ls_ref[...] += jnp.sum(diff * diff).reshape(1, 1)


def _stage_call(first, final, n, dim, num_codes, b):
    body = functools.partial(_stage_body, first, final)
    row = lambda i: (i, 0)
    rep = lambda i: (0, 0)
    row_spec = pl.BlockSpec((b, dim), row)
    cb_spec = pl.BlockSpec((num_codes, dim), rep)
    in_specs = []
    if final:
        in_specs.append(row_spec)
    in_specs.append(row_spec)
    if not first:
        in_specs.append(row_spec)
    in_specs += [cb_spec,
                 pl.BlockSpec((3 * num_codes, dim), rep),
                 pl.BlockSpec((b, 1), row),
                 pl.BlockSpec((1, num_codes), rep)]
    out_specs = [pl.BlockSpec((b, 1), row)]
    out_shape = [jax.ShapeDtypeStruct((n, 1), jnp.int32)]
    if not final:
        out_specs.append(row_spec)
        out_shape.append(jax.ShapeDtypeStruct((n, dim), jnp.float32))
    out_specs.append(row_spec)
    out_shape.append(jax.ShapeDtypeStruct((n, dim), jnp.float32))
    out_specs.append(pl.BlockSpec((1, 1), rep))
    out_shape.append(jax.ShapeDtypeStruct((1, 1), jnp.float32))
    return pl.pallas_call(body, grid=(n // b,), in_specs=in_specs,
                          out_specs=out_specs, out_shape=out_shape)


def kernel(embeds, codebooks):
    n, dim = embeds.shape
    depth, num_codes, _ = codebooks.shape
    b = _BLOCK

    h1 = jax.lax.reduce_precision(codebooks, 8, 7)
    c1 = codebooks - h1
    h2 = jax.lax.reduce_precision(c1, 8, 7)
    h3 = c1 - h2
    hcat = jnp.concatenate([h1, h2, h3], axis=1).astype(jnp.bfloat16)

    r = embeds
    qs = None
    idx_cols = []
    losses = []
    for g in range(depth):
        cb = codebooks[g]
        r2 = jnp.sum(r * r, axis=1, keepdims=True)
        cb2 = jnp.sum(cb * cb, axis=1)[None, :]
        first = g == 0
        final = g == depth - 1
        call = _stage_call(first, final, n, dim, num_codes, b)
        args = []
        if final:
            args.append(embeds)
        args.append(r)
        if not first:
            args.append(qs)
        args += [cb, hcat[g], r2, cb2]
        outs = call(*args)
        if final:
            idx, qs, ls = outs
        else:
            idx, r, qs, ls = outs
        idx_cols.append(idx)
        m = ls[0, 0] / (n * dim)
        losses.append(m + 0.25 * m)
    indices = jnp.concatenate(idx_cols, axis=1)
    loss = jnp.mean(jnp.stack(losses))
    return qs, indices, loss


# speedup vs baseline: 1.0988x; 1.0685x over previous
import functools
import jax
import jax.numpy as jnp
from jax.experimental import pallas as pl

_BLOCK = 1024


def _stage_body(first, final, *refs):
    if first:
        (r_ref, cb_ref, h_ref, r2_ref, cb2_ref,
         idx_ref, rn_ref, qn_ref, ls_ref) = refs
        e_ref = qs_ref = None
    elif final:
        (e_ref, r_ref, qs_ref, cb_ref, h_ref, r2_ref,
         cb2_ref, idx_ref, qn_ref, ls_ref) = refs
        rn_ref = None
    else:
        (r_ref, qs_ref, cb_ref, h_ref, r2_ref, cb2_ref,
         idx_ref, rn_ref, qn_ref, ls_ref) = refs
    i = pl.program_id(0)
    num_codes = cb_ref.shape[0]
    r = r_ref[...]
    cb = cb_ref[...]
    s = jax.lax.dot_general(r, cb, (((1,), (1,)), ((), ())),
                            preferred_element_type=jnp.float32)
    d = r2_ref[...] - 2.0 * s + cb2_ref[...]
    md = jnp.min(d, axis=1, keepdims=True)
    iota = jax.lax.broadcasted_iota(jnp.int32, d.shape, 1)
    idx = jnp.min(jnp.where(d == md, iota, num_codes), axis=1, keepdims=True)
    idx_ref[...] = idx
    oh = (iota == idx).astype(jnp.bfloat16)
    dn = (((1,), (0,)), ((), ()))
    oh3 = jnp.concatenate([oh, oh, oh], axis=1)
    q = jax.lax.dot_general(oh3, h_ref[...], dn,
                            preferred_element_type=jnp.float32)
    diff = q - r
    qs = q if first else qs_ref[...] + q
    if final:
        e = e_ref[...]
        qn_ref[...] = e + (qs - e)
    else:
        qn_ref[...] = qs
    if not final:
        rn_ref[...] = r - q

    @pl.when(i == 0)
    def _():
        ls_ref[...] = jnp.zeros_like(ls_ref)

    ls_ref[...] += jnp.sum(diff * diff).reshape(1, 1)


def _stage_call(first, final, n, dim, num_codes, b):
    body = functools.partial(_stage_body, first, final)
    row = lambda i: (i, 0)
    rep = lambda i: (0, 0)
    row_spec = pl.BlockSpec((b, dim), row)
    cb_spec = pl.BlockSpec((num_codes, dim), rep)
    in_specs = []
    if final:
        in_specs.append(row_spec)
    in_specs.append(row_spec)
    if not first:
        in_specs.append(row_spec)
    in_specs += [cb_spec,
                 pl.BlockSpec((3 * num_codes, dim), rep),
                 pl.BlockSpec((b, 1), row),
                 pl.BlockSpec((1, num_codes), rep)]
    out_specs = [pl.BlockSpec((b, 1), row)]
    out_shape = [jax.ShapeDtypeStruct((n, 1), jnp.int32)]
    if not final:
        out_specs.append(row_spec)
        out_shape.append(jax.ShapeDtypeStruct((n, dim), jnp.float32))
    out_specs.append(row_spec)
    out_shape.append(jax.ShapeDtypeStruct((n, dim), jnp.float32))
    out_specs.append(pl.BlockSpec((1, 1), rep))
    out_shape.append(jax.ShapeDtypeStruct((1, 1), jnp.float32))
    return pl.pallas_call(body, grid=(n // b,), in_specs=in_specs,
                          out_specs=out_specs, out_shape=out_shape)


def kernel(embeds, codebooks):
    n, dim = embeds.shape
    depth, num_codes, _ = codebooks.shape
    b = _BLOCK

    h1 = jax.lax.reduce_precision(codebooks, 8, 7)
    c1 = codebooks - h1
    h2 = jax.lax.reduce_precision(c1, 8, 7)
    h3 = c1 - h2
    hcat = jnp.concatenate([h1, h2, h3], axis=1).astype(jnp.bfloat16)

    r = embeds
    qs = None
    idx_cols = []
    losses = []
    for g in range(depth):
        cb = codebooks[g]
        r2 = jnp.sum(r * r, axis=1, keepdims=True)
        cb2 = jnp.sum(cb * cb, axis=1)[None, :]
        first = g == 0
        final = g == depth - 1
        call = _stage_call(first, final, n, dim, num_codes, b)
        args = []
        if final:
            args.append(embeds)
        args.append(r)
        if not first:
            args.append(qs)
        args += [cb, hcat[g], r2, cb2]
        outs = call(*args)
        if final:
            idx, qs, ls = outs
        else:
            idx, r, qs, ls = outs
        idx_cols.append(idx)
        m = ls[0, 0] / (n * dim)
        losses.append(m + 0.25 * m)
    indices = jnp.concatenate(idx_cols, axis=1)
    loss = jnp.mean(jnp.stack(losses))
    return qs, indices, loss


# speedup vs baseline: 1.1007x; 1.0018x over previous
import functools
import jax
import jax.numpy as jnp
from jax.experimental import pallas as pl

_BLOCK = 1536


def _stage_body(first, final, *refs):
    if first:
        (r_ref, cb_ref, h_ref, r2_ref, cb2_ref,
         idx_ref, rn_ref, qn_ref, ls_ref) = refs
        e_ref = qs_ref = None
    elif final:
        (e_ref, r_ref, qs_ref, cb_ref, h_ref, r2_ref,
         cb2_ref, idx_ref, qn_ref, ls_ref) = refs
        rn_ref = None
    else:
        (r_ref, qs_ref, cb_ref, h_ref, r2_ref, cb2_ref,
         idx_ref, rn_ref, qn_ref, ls_ref) = refs
    i = pl.program_id(0)
    num_codes = cb_ref.shape[0]
    r = r_ref[...]
    cb = cb_ref[...]
    s = jax.lax.dot_general(r, cb, (((1,), (1,)), ((), ())),
                            preferred_element_type=jnp.float32)
    d = r2_ref[...] - 2.0 * s + cb2_ref[...]
    md = jnp.min(d, axis=1, keepdims=True)
    iota = jax.lax.broadcasted_iota(jnp.int32, d.shape, 1)
    idx = jnp.min(jnp.where(d == md, iota, num_codes), axis=1, keepdims=True)
    idx_ref[...] = idx
    oh = (iota == idx).astype(jnp.bfloat16)
    dn = (((1,), (0,)), ((), ()))
    oh3 = jnp.concatenate([oh, oh, oh], axis=1)
    q = jax.lax.dot_general(oh3, h_ref[...], dn,
                            preferred_element_type=jnp.float32)
    diff = q - r
    qs = q if first else qs_ref[...] + q
    if final:
        e = e_ref[...]
        qn_ref[...] = e + (qs - e)
    else:
        qn_ref[...] = qs
    if not final:
        rn_ref[...] = r - q

    @pl.when(i == 0)
    def _():
        ls_ref[...] = jnp.zeros_like(ls_ref)

    ls_ref[...] += jnp.sum(diff * diff).reshape(1, 1)


def _stage_call(first, final, n, dim, num_codes, b):
    body = functools.partial(_stage_body, first, final)
    row = lambda i: (i, 0)
    rep = lambda i: (0, 0)
    row_spec = pl.BlockSpec((b, dim), row)
    cb_spec = pl.BlockSpec((num_codes, dim), rep)
    in_specs = []
    if final:
        in_specs.append(row_spec)
    in_specs.append(row_spec)
    if not first:
        in_specs.append(row_spec)
    in_specs += [cb_spec,
                 pl.BlockSpec((3 * num_codes, dim), rep),
                 pl.BlockSpec((b, 1), row),
                 pl.BlockSpec((1, num_codes), rep)]
    out_specs = [pl.BlockSpec((b, 1), row)]
    out_shape = [jax.ShapeDtypeStruct((n, 1), jnp.int32)]
    if not final:
        out_specs.append(row_spec)
        out_shape.append(jax.ShapeDtypeStruct((n, dim), jnp.float32))
    out_specs.append(row_spec)
    out_shape.append(jax.ShapeDtypeStruct((n, dim), jnp.float32))
    out_specs.append(pl.BlockSpec((1, 1), rep))
    out_shape.append(jax.ShapeDtypeStruct((1, 1), jnp.float32))
    return pl.pallas_call(body, grid=(n // b,), in_specs=in_specs,
                          out_specs=out_specs, out_shape=out_shape)


def kernel(embeds, codebooks):
    n, dim = embeds.shape
    depth, num_codes, _ = codebooks.shape
    b = _BLOCK

    h1 = jax.lax.reduce_precision(codebooks, 8, 7)
    c1 = codebooks - h1
    h2 = jax.lax.reduce_precision(c1, 8, 7)
    h3 = c1 - h2
    hcat = jnp.concatenate([h1, h2, h3], axis=1).astype(jnp.bfloat16)

    r = embeds
    qs = None
    idx_cols = []
    losses = []
    for g in range(depth):
        cb = codebooks[g]
        r2 = jnp.sum(r * r, axis=1, keepdims=True)
        cb2 = jnp.sum(cb * cb, axis=1)[None, :]
        first = g == 0
        final = g == depth - 1
        call = _stage_call(first, final, n, dim, num_codes, b)
        args = []
        if final:
            args.append(embeds)
        args.append(r)
        if not first:
            args.append(qs)
        args += [cb, hcat[g], r2, cb2]
        outs = call(*args)
        if final:
            idx, qs, ls = outs
        else:
            idx, r, qs, ls = outs
        idx_cols.append(idx)
        m = ls[0, 0] / (n * dim)
        losses.append(m + 0.25 * m)
    indices = jnp.concatenate(idx_cols, axis=1)
    loss = jnp.mean(jnp.stack(losses))
    return qs, indices, loss
